# speedup vs baseline: 2.1860x; 2.1860x over previous
import functools

import jax
import jax.numpy as jnp
from jax import lax
from jax.experimental import pallas as pl
from jax.experimental.pallas import tpu as pltpu
from jax.experimental.pallas import tpu_sc as plsc

_EMBED = 128
_MAX_STEPS = 1000
_PAD_STEPS = 1024


def _build_table(dim, pad_steps):
    steps = jnp.arange(pad_steps, dtype=jnp.float32)[:, None]
    dims = jnp.arange(dim, dtype=jnp.float32)[None, :]
    t = steps * 10.0 ** (dims * 4.0 / dim)
    return jnp.concatenate([jnp.sin(t), jnp.cos(t)], axis=1)


def _mlp_body(table_ref, w1_ref, b1_ref, w2_ref, b2_ref, out_ref):
    h = jnp.dot(table_ref[...], w1_ref[...],
                preferred_element_type=jnp.float32,
                precision=lax.Precision.HIGHEST) + b1_ref[...]
    h = h * jax.nn.sigmoid(h)
    o = jnp.dot(h, w2_ref[...],
                preferred_element_type=jnp.float32,
                precision=lax.Precision.HIGHEST) + b2_ref[...]
    out_ref[...] = o * jax.nn.sigmoid(o)


def _mlp_on_table(table, W1, b1, W2, b2):
    return pl.pallas_call(
        _mlp_body,
        out_shape=jax.ShapeDtypeStruct((_PAD_STEPS, _EMBED), jnp.float32),
    )(table, W1, b1.reshape(1, _EMBED), W2, b2.reshape(1, _EMBED))


@functools.lru_cache(maxsize=None)
def _make_sc_gather(batch):
    info = plsc.get_sparse_core_info()
    nw = info.num_cores * info.num_subcores
    b_per_w = batch // nw
    chunk = 128
    n_chunks = b_per_w // chunk
    mesh = plsc.VectorSubcoreMesh(core_axis_name="c", subcore_axis_name="s")

    @functools.partial(
        pl.kernel,
        mesh=mesh,
        out_type=jax.ShapeDtypeStruct((batch, _EMBED), jnp.float32),
        scratch_types=[
            pltpu.VMEM((n_chunks, chunk), jnp.int32),
            pltpu.VMEM((b_per_w, _EMBED), jnp.float32),
            pltpu.SemaphoreType.DMA,
        ],
    )
    def gather(table_hbm, idx_hbm, out_hbm, idx_v, rows_v, sem):
        wid = lax.axis_index("s") * info.num_cores + lax.axis_index("c")
        pltpu.sync_copy(idx_hbm.at[wid], idx_v)
        copies = []
        for j in range(n_chunks):
            copies.append(
                pltpu.async_copy(table_hbm.at[idx_v.at[j]],
                                 rows_v.at[pl.ds(j * chunk, chunk)], sem))
        for c in copies:
            c.wait()
        pltpu.sync_copy(rows_v, out_hbm.at[pl.ds(wid * b_per_w, b_per_w)])

    return gather, nw, n_chunks, chunk


def kernel(diffusion_step, W1, b1, W2, b2):
    table = _build_table(_EMBED, _PAD_STEPS)
    out_table = _mlp_on_table(table, W1, b1, W2, b2)
    gather, nw, n_chunks, chunk = _make_sc_gather(diffusion_step.shape[0])
    idx = diffusion_step.reshape(nw, n_chunks, chunk)
    return gather(out_table, idx)


# speedup vs baseline: 2.2925x; 1.0487x over previous
import functools

import jax
import jax.numpy as jnp
from jax import lax
from jax.experimental import pallas as pl
from jax.experimental.pallas import tpu as pltpu
from jax.experimental.pallas import tpu_sc as plsc

_EMBED = 128
_MAX_STEPS = 1000
_PAD_STEPS = 1024


@functools.lru_cache(maxsize=None)
def _build_table(dim, pad_steps):
    steps = jnp.arange(pad_steps, dtype=jnp.float32)[:, None]
    dims = jnp.arange(dim, dtype=jnp.float32)[None, :]
    t = steps * 10.0 ** (dims * 4.0 / dim)
    out = jnp.concatenate([jnp.sin(t), jnp.cos(t)], axis=1)
    return jax.block_until_ready(out)


def _mlp_body(table_ref, w1_ref, b1_ref, w2_ref, b2_ref, out_ref):
    h = jnp.dot(table_ref[...], w1_ref[...],
                preferred_element_type=jnp.float32) + b1_ref[...]
    h = h * jax.nn.sigmoid(h)
    o = jnp.dot(h, w2_ref[...],
                preferred_element_type=jnp.float32) + b2_ref[...]
    out_ref[...] = o * jax.nn.sigmoid(o)


def _mlp_on_table(table, W1, b1, W2, b2):
    return pl.pallas_call(
        _mlp_body,
        out_shape=jax.ShapeDtypeStruct((_PAD_STEPS, _EMBED), jnp.float32),
    )(table, W1, b1.reshape(1, _EMBED), W2, b2.reshape(1, _EMBED))


@functools.lru_cache(maxsize=None)
def _make_sc_gather(batch):
    info = plsc.get_sparse_core_info()
    nw = info.num_cores * info.num_subcores
    b_per_w = batch // nw
    chunk = 128
    n_chunks = b_per_w // chunk
    mesh = plsc.VectorSubcoreMesh(core_axis_name="c", subcore_axis_name="s")

    @functools.partial(
        pl.kernel,
        mesh=mesh,
        out_type=jax.ShapeDtypeStruct((batch, _EMBED), jnp.float32),
        scratch_types=[
            pltpu.VMEM((n_chunks, chunk), jnp.int32),
            pltpu.VMEM((b_per_w, _EMBED), jnp.float32),
        ] + [pltpu.SemaphoreType.DMA] * (n_chunks + 1),
    )
    def gather(table_hbm, idx_hbm, out_hbm, idx_v, rows_v, *sems):
        gsems, wsem = sems[:n_chunks], sems[n_chunks]
        wid = lax.axis_index("s") * info.num_cores + lax.axis_index("c")
        pltpu.sync_copy(idx_hbm.at[wid], idx_v)
        gathers = []
        for j in range(n_chunks):
            gathers.append(
                pltpu.async_copy(table_hbm.at[idx_v.at[j]],
                                 rows_v.at[pl.ds(j * chunk, chunk)], gsems[j]))
        writes = []
        base = wid * b_per_w
        for j in range(n_chunks):
            gathers[j].wait()
            writes.append(
                pltpu.async_copy(rows_v.at[pl.ds(j * chunk, chunk)],
                                 out_hbm.at[pl.ds(base + j * chunk, chunk)],
                                 wsem))
        for w in writes:
            w.wait()

    return gather, nw, n_chunks, chunk


def kernel(diffusion_step, W1, b1, W2, b2):
    table = _build_table(_EMBED, _PAD_STEPS)
    out_table = _mlp_on_table(table, W1, b1, W2, b2)
    gather, nw, n_chunks, chunk = _make_sc_gather(diffusion_step.shape[0])
    idx = diffusion_step.reshape(nw, n_chunks, chunk)
    return gather(out_table, idx)


# speedup vs baseline: 2.6749x; 1.1668x over previous
import functools

import jax
import jax.numpy as jnp
from jax import lax
from jax.experimental import pallas as pl
from jax.experimental.pallas import tpu as pltpu
from jax.experimental.pallas import tpu_sc as plsc

_EMBED = 128
_MAX_STEPS = 1000
_PAD_STEPS = 1024


def _build_table(dim, pad_steps):
    steps = jnp.arange(pad_steps, dtype=jnp.float32)[:, None]
    dims = jnp.arange(dim, dtype=jnp.float32)[None, :]
    t = steps * 10.0 ** (dims * 4.0 / dim)
    return jnp.concatenate([jnp.sin(t), jnp.cos(t)], axis=1)


try:
    _TABLE_CONST = jax.block_until_ready(_build_table(_EMBED, _PAD_STEPS))
except Exception:
    _TABLE_CONST = None


def _mlp_body(table_ref, w1_ref, b1_ref, w2_ref, b2_ref, out_ref):
    h = jnp.dot(table_ref[...], w1_ref[...],
                preferred_element_type=jnp.float32) + b1_ref[...]
    h = h * jax.nn.sigmoid(h)
    o = jnp.dot(h, w2_ref[...],
                preferred_element_type=jnp.float32) + b2_ref[...]
    out_ref[...] = o * jax.nn.sigmoid(o)


def _mlp_on_table(table, W1, b1, W2, b2):
    return pl.pallas_call(
        _mlp_body,
        out_shape=jax.ShapeDtypeStruct((_PAD_STEPS, _EMBED), jnp.float32),
    )(table, W1, b1.reshape(1, _EMBED), W2, b2.reshape(1, _EMBED))


@functools.lru_cache(maxsize=None)
def _make_sc_gather(batch):
    info = plsc.get_sparse_core_info()
    nw = info.num_cores * info.num_subcores
    b_per_w = batch // nw
    chunk = 128
    n_chunks = b_per_w // chunk
    mesh = plsc.VectorSubcoreMesh(core_axis_name="c", subcore_axis_name="s")

    @functools.partial(
        pl.kernel,
        mesh=mesh,
        out_type=jax.ShapeDtypeStruct((batch, _EMBED), jnp.float32),
        scratch_types=[
            pltpu.VMEM((n_chunks, chunk), jnp.int32),
            pltpu.VMEM((b_per_w, _EMBED), jnp.float32),
        ] + [pltpu.SemaphoreType.DMA] * (n_chunks + 1),
    )
    def gather(table_hbm, idx_hbm, out_hbm, idx_v, rows_v, *sems):
        gsems, wsem = sems[:n_chunks], sems[n_chunks]
        wid = lax.axis_index("s") * info.num_cores + lax.axis_index("c")
        pltpu.sync_copy(idx_hbm.at[wid], idx_v)
        base = wid * b_per_w
        gathers = []
        for j in range(n_chunks):
            gathers.append(
                pltpu.async_copy(table_hbm.at[idx_v.at[j]],
                                 rows_v.at[pl.ds(j * chunk, chunk)], gsems[j]))
        writes = []
        for j in range(n_chunks):
            gathers[j].wait()
            writes.append(
                pltpu.async_copy(rows_v.at[pl.ds(j * chunk, chunk)],
                                 out_hbm.at[pl.ds(base + j * chunk, chunk)],
                                 wsem))
        for w in writes:
            w.wait()

    return gather, nw, n_chunks, chunk


def kernel(diffusion_step, W1, b1, W2, b2):
    table = (_TABLE_CONST if _TABLE_CONST is not None
             else _build_table(_EMBED, _PAD_STEPS))
    out_table = _mlp_on_table(table, W1, b1, W2, b2)
    gather, nw, n_chunks, chunk = _make_sc_gather(diffusion_step.shape[0])
    idx = diffusion_step.reshape(nw, n_chunks, chunk)
    return gather(out_table, idx)


# speedup vs baseline: 3.0507x; 1.1405x over previous
import functools

import jax
import jax.numpy as jnp
from jax import lax
from jax.experimental import pallas as pl
from jax.experimental.pallas import tpu as pltpu
from jax.experimental.pallas import tpu_sc as plsc

_EMBED = 128
_MAX_STEPS = 1000
_PAD_STEPS = 1024


def _build_table(dim, pad_steps):
    steps = jnp.arange(pad_steps, dtype=jnp.float32)[:, None]
    dims = jnp.arange(dim, dtype=jnp.float32)[None, :]
    t = steps * 10.0 ** (dims * 4.0 / dim)
    return jnp.concatenate([jnp.sin(t), jnp.cos(t)], axis=1)


try:
    _TABLE_CONST = jax.block_until_ready(_build_table(_EMBED, _PAD_STEPS))
except Exception:
    _TABLE_CONST = None


def _mlp_body(table_ref, w1_ref, b1_ref, w2_ref, b2_ref, out_ref):
    h = jnp.dot(table_ref[...], w1_ref[...],
                preferred_element_type=jnp.float32) + b1_ref[...]
    h = h * jax.nn.sigmoid(h)
    o = jnp.dot(h, w2_ref[...],
                preferred_element_type=jnp.float32) + b2_ref[...]
    out_ref[...] = o * jax.nn.sigmoid(o)


def _mlp_on_table(table, W1, b1, W2, b2):
    return pl.pallas_call(
        _mlp_body,
        out_shape=jax.ShapeDtypeStruct((_PAD_STEPS, _EMBED), jnp.float32),
    )(table, W1, b1.reshape(1, _EMBED), W2, b2.reshape(1, _EMBED))


@functools.lru_cache(maxsize=None)
def _make_sc_gather(batch):
    info = plsc.get_sparse_core_info()
    nw = info.num_cores * info.num_subcores
    b_per_w = batch // nw
    chunk = 128
    n_chunks = b_per_w // chunk
    mesh = plsc.VectorSubcoreMesh(core_axis_name="c", subcore_axis_name="s")

    @functools.partial(
        pl.kernel,
        mesh=mesh,
        out_type=jax.ShapeDtypeStruct((batch, _EMBED), jnp.float32),
        scratch_types=[
            pltpu.VMEM((n_chunks, chunk), jnp.int32),
            pltpu.VMEM((b_per_w, _EMBED), jnp.float32),
            pltpu.VMEM_SHARED((_PAD_STEPS, _EMBED), jnp.float32),
        ] + [pltpu.SemaphoreType.DMA] * (n_chunks + 1),
    )
    def gather(table_hbm, idx_hbm, out_hbm, idx_v, rows_v, table_sp, *sems):
        gsems, wsem = sems[:n_chunks], sems[n_chunks]
        sid = lax.axis_index("s")
        wid = sid * info.num_cores + lax.axis_index("c")
        @pl.when(sid == 0)
        def _():
            pltpu.sync_copy(table_hbm, table_sp)
        pltpu.sync_copy(idx_hbm.at[wid], idx_v)
        plsc.subcore_barrier()
        base = wid * b_per_w
        gathers = []
        for j in range(n_chunks):
            gathers.append(
                pltpu.async_copy(table_sp.at[idx_v.at[j]],
                                 rows_v.at[pl.ds(j * chunk, chunk)], gsems[j]))
        writes = []
        for j in range(n_chunks):
            gathers[j].wait()
            writes.append(
                pltpu.async_copy(rows_v.at[pl.ds(j * chunk, chunk)],
                                 out_hbm.at[pl.ds(base + j * chunk, chunk)],
                                 wsem))
        for w in writes:
            w.wait()

    return gather, nw, n_chunks, chunk


def kernel(diffusion_step, W1, b1, W2, b2):
    table = (_TABLE_CONST if _TABLE_CONST is not None
             else _build_table(_EMBED, _PAD_STEPS))
    out_table = _mlp_on_table(table, W1, b1, W2, b2)
    gather, nw, n_chunks, chunk = _make_sc_gather(diffusion_step.shape[0])
    idx = diffusion_step.reshape(nw, n_chunks, chunk)
    return gather(out_table, idx)
